# Initial kernel scaffold; baseline (speedup 1.0000x reference)
import jax
import jax.numpy as jnp
from jax.experimental import pallas as pl


def kernel(x, edge_index, W1, b1, a1, g1, be1, W2, b2, a2, g2, be2, W3, b3, a3, g3, be3, W4, b4):
    raise NotImplementedError("write your pallas kernel here")


# speedup vs baseline: 12.1806x; 12.1806x over previous
import functools

import jax
import jax.numpy as jnp
from jax import lax
from jax.experimental import pallas as pl
from jax.experimental.pallas import tpu as pltpu
from jax.experimental.pallas import tpu_sc as plsc

_N = 10000
_E = 320000
_F = 128
_NC = 2
_NS = 16
_NW = _NC * _NS
_NPAD = 10240
_RPT = _NPAD // _NS
_EPT = _E // _NW
_CHUNK = 128
_NFULL = _EPT // _CHUNK
_REM = _EPT - _NFULL * _CHUNK


def _sc_mesh():
    return plsc.VectorSubcoreMesh(
        core_axis_name="c", subcore_axis_name="s",
        num_cores=_NC, num_subcores=_NS)


def _deg_hists(dst):
    ones = jnp.ones((_CHUNK, _F), jnp.float32)
    zeros = jnp.zeros((_RPT, _F), jnp.float32)

    @functools.partial(
        pl.kernel,
        out_type=jax.ShapeDtypeStruct((_NC, _NPAD, _F), jnp.float32),
        mesh=_sc_mesh(),
        scratch_types=[
            pltpu.VMEM((_CHUNK,), jnp.int32),
            pltpu.VMEM((_REM,), jnp.int32),
            pltpu.VMEM((_CHUNK, _F), jnp.float32),
            pltpu.VMEM_SHARED((_NPAD, _F), jnp.float32),
        ],
    )
    def deg_k(dst_hbm, ones_hbm, zeros_hbm, hist_hbm, dst_v, dst_r, ones_v,
              hist):
        c = lax.axis_index("c")
        s = lax.axis_index("s")
        wid = c * _NS + s
        r0 = s * _RPT

        pltpu.sync_copy(zeros_hbm, hist.at[pl.ds(r0, _RPT)])
        pltpu.sync_copy(ones_hbm, ones_v)
        plsc.subcore_barrier()

        e0 = wid * _EPT

        def cbody(j, carry):
            pltpu.sync_copy(dst_hbm.at[pl.ds(e0 + j * _CHUNK, _CHUNK)], dst_v)
            pltpu.sync_copy(ones_v, hist.at[dst_v], add=True)
            return carry
        lax.fori_loop(0, _NFULL, cbody, 0)

        pltpu.sync_copy(dst_hbm.at[pl.ds(e0 + _NFULL * _CHUNK, _REM)], dst_r)
        pltpu.sync_copy(ones_v.at[pl.ds(0, _REM)], hist.at[dst_r], add=True)

        plsc.subcore_barrier()
        pltpu.sync_copy(hist.at[pl.ds(r0, _RPT)],
                        hist_hbm.at[c, pl.ds(r0, _RPT)])

    return deg_k(dst, ones, zeros)


def _spmm(table, src, dst):
    zeros = jnp.zeros((_RPT, _F), jnp.float32)

    @functools.partial(
        pl.kernel,
        out_type=jax.ShapeDtypeStruct((_NC, _NPAD, _F), jnp.float32),
        mesh=_sc_mesh(),
        scratch_types=[
            pltpu.VMEM((_CHUNK,), jnp.int32),
            pltpu.VMEM((_CHUNK,), jnp.int32),
            pltpu.VMEM((_REM,), jnp.int32),
            pltpu.VMEM((_REM,), jnp.int32),
            pltpu.VMEM((_CHUNK, _F), jnp.float32),
            pltpu.VMEM((_REM, _F), jnp.float32),
            pltpu.VMEM_SHARED((_NPAD, _F), jnp.float32),
            pltpu.SemaphoreType.DMA,
        ],
    )
    def spmm_k(tab_hbm, src_hbm, dst_hbm, zeros_hbm, out_hbm,
               src_v, dst_v, src_r, dst_r, rows_v, rows_r, acc, sem):
        c = lax.axis_index("c")
        s = lax.axis_index("s")
        wid = c * _NS + s

        r0 = s * _RPT
        pltpu.sync_copy(zeros_hbm, acc.at[pl.ds(r0, _RPT)])
        plsc.subcore_barrier()

        e0 = wid * _EPT

        def cbody(j, carry):
            b = e0 + j * _CHUNK
            pltpu.sync_copy(src_hbm.at[pl.ds(b, _CHUNK)], src_v)
            pltpu.async_copy(tab_hbm.at[src_v], rows_v, sem).wait()
            pltpu.sync_copy(dst_hbm.at[pl.ds(b, _CHUNK)], dst_v)
            pltpu.sync_copy(rows_v, acc.at[dst_v], add=True)
            return carry
        lax.fori_loop(0, _NFULL, cbody, 0)

        b = e0 + _NFULL * _CHUNK
        pltpu.sync_copy(src_hbm.at[pl.ds(b, _REM)], src_r)
        pltpu.async_copy(tab_hbm.at[src_r], rows_r, sem).wait()
        pltpu.sync_copy(dst_hbm.at[pl.ds(b, _REM)], dst_r)
        pltpu.sync_copy(rows_r, acc.at[dst_r], add=True)

        plsc.subcore_barrier()
        pltpu.sync_copy(acc.at[pl.ds(r0, _RPT)],
                        out_hbm.at[c, pl.ds(r0, _RPT)])

    return spmm_k(table, src, dst, zeros)


def _tc_pre(hists, x, w1):
    def body(h_ref, x_ref, w_ref, dis_ref, g1_ref):
        deg = h_ref[:_N, 0:1] + h_ref[_NPAD:_NPAD + _N, 0:1]
        dis = lax.rsqrt(deg + 1.0)
        dis_ref[...] = dis
        g1_ref[...] = dis * jnp.dot(x_ref[...], w_ref[...],
                                    preferred_element_type=jnp.float32)

    return pl.pallas_call(
        body,
        out_shape=(jax.ShapeDtypeStruct((_N, 1), jnp.float32),
                   jax.ShapeDtypeStruct((_N, _F), jnp.float32)),
    )(hists, x, w1)


def _tc_layer(p, g_prev, dis, b, a, gam, bet, w_next):
    def body(p_ref, g_ref, dis_ref, b_ref, a_ref, gam_ref, bet_ref, w_ref,
             out_ref):
        dis = dis_ref[...]
        ssum = p_ref[0, :_N, :] + p_ref[1, :_N, :] + g_ref[...]
        conv = dis * ssum + b_ref[...]
        h = conv * jax.nn.sigmoid(a_ref[...])
        mu = jnp.mean(h, axis=1, keepdims=True)
        d = h - mu
        var = jnp.mean(d * d, axis=1, keepdims=True)
        hn = d * lax.rsqrt(var + 1e-5) * gam_ref[...] + bet_ref[...]
        he = jnp.where(hn > 0, hn, jnp.exp(jnp.minimum(hn, 0.0)) - 1.0)
        out_ref[...] = dis * jnp.dot(he, w_ref[...],
                                     preferred_element_type=jnp.float32)

    return pl.pallas_call(
        body,
        out_shape=jax.ShapeDtypeStruct((_N, _F), jnp.float32),
    )(p, g_prev, dis, b, a, gam, bet, w_next)


def _tc_final(p, g_last, dis, b):
    def body(p_ref, g_ref, dis_ref, b_ref, out_ref):
        ssum = p_ref[0, :_N, :] + p_ref[1, :_N, :] + g_ref[...]
        out_ref[...] = dis_ref[...] * ssum + b_ref[...]

    return pl.pallas_call(
        body,
        out_shape=jax.ShapeDtypeStruct((_N, _F), jnp.float32),
    )(p, g_last, dis, b)


def kernel(x, edge_index, W1, b1, a1, g1, be1, W2, b2, a2, g2, be2,
           W3, b3, a3, g3, be3, W4, b4):
    src = edge_index[0]
    dst = edge_index[1]

    hists = _deg_hists(dst).reshape(_NC * _NPAD, _F)
    dis, h = _tc_pre(hists, x, W1)

    layer_params = [
        (b1, a1, g1, be1, W2),
        (b2, a2, g2, be2, W3),
        (b3, a3, g3, be3, W4),
    ]
    for (b, a, gam, bet, w_next) in layer_params:
        p = _spmm(h, src, dst)
        h = _tc_layer(p, h, dis, b.reshape(1, _F), a.reshape(1, 1),
                      gam.reshape(1, _F), bet.reshape(1, _F), w_next)

    p = _spmm(h, src, dst)
    return _tc_final(p, h, dis, b4.reshape(1, _F))


# speedup vs baseline: 12.8174x; 1.0523x over previous
import functools

import jax
import jax.numpy as jnp
from jax import lax
from jax.experimental import pallas as pl
from jax.experimental.pallas import tpu as pltpu
from jax.experimental.pallas import tpu_sc as plsc

_N = 10000
_E = 320000
_F = 128
_NC = 2
_NS = 16
_NW = _NC * _NS
_NPAD = 10240
_RPT = _NPAD // _NS
_EPT = _E // _NW
_CHUNK = 128
_NFULL = _EPT // _CHUNK
_NCH = _NFULL + 1
_IPT = _NCH * _CHUNK


def _sc_mesh():
    return plsc.VectorSubcoreMesh(
        core_axis_name="c", subcore_axis_name="s",
        num_cores=_NC, num_subcores=_NS)


def _fill_dst2d(dst_hbm, junk_hbm, dst2d, wid, sem):
    e0 = wid * _EPT
    pltpu.sync_copy(junk_hbm, dst2d.at[_NFULL])

    def rbody(j, carry):
        pltpu.async_copy(dst_hbm.at[pl.ds(e0 + j * _CHUNK, _CHUNK)],
                         dst2d.at[j], sem)
        return carry
    lax.fori_loop(0, _NFULL, rbody, 0)
    pltpu.async_copy(dst_hbm.at[pl.ds(e0 + _NFULL * _CHUNK, 16)],
                     dst2d.at[_NFULL, pl.ds(0, 16)], sem)
    def dbody(j, carry):
        pltpu.make_async_copy(dst_hbm.at[pl.ds(0, _CHUNK)],
                              dst2d.at[j], sem).wait()
        return carry
    lax.fori_loop(0, _NFULL, dbody, 0)
    pltpu.make_async_copy(dst_hbm.at[pl.ds(0, 16)],
                          dst2d.at[_NFULL, pl.ds(0, 16)], sem).wait()


def _deg_hists(dst):
    ones = jnp.ones((_CHUNK, _F), jnp.float32)
    zeros = jnp.zeros((_RPT, _F), jnp.float32)
    junk = jnp.full((_CHUNK,), _NPAD - 1, jnp.int32)

    @functools.partial(
        pl.kernel,
        out_type=jax.ShapeDtypeStruct((_NC, _NPAD, _F), jnp.float32),
        mesh=_sc_mesh(),
        scratch_types=[
            pltpu.VMEM((_NCH, _CHUNK), jnp.int32),
            pltpu.VMEM((_CHUNK, _F), jnp.float32),
            pltpu.VMEM_SHARED((_NPAD, _F), jnp.float32),
            pltpu.SemaphoreType.DMA,
        ],
    )
    def deg_k(dst_hbm, ones_hbm, zeros_hbm, junk_hbm, hist_hbm,
              dst2d, ones_v, hist, sem):
        c = lax.axis_index("c")
        s = lax.axis_index("s")
        wid = c * _NS + s
        r0 = s * _RPT

        pltpu.sync_copy(zeros_hbm, hist.at[pl.ds(r0, _RPT)])
        pltpu.sync_copy(ones_hbm, ones_v)
        _fill_dst2d(dst_hbm, junk_hbm, dst2d, wid, sem)
        plsc.subcore_barrier()

        def cbody(j, carry):
            pltpu.sync_copy(ones_v, hist.at[dst2d.at[j]], add=True)
            return carry
        lax.fori_loop(0, _NCH, cbody, 0)

        plsc.subcore_barrier()
        pltpu.sync_copy(hist.at[pl.ds(r0, _RPT)],
                        hist_hbm.at[c, pl.ds(r0, _RPT)])

    return deg_k(dst, ones, zeros, junk)


def _spmm(table, src, dst):
    zeros = jnp.zeros((_RPT, _F), jnp.float32)
    zidx = jnp.zeros((_CHUNK,), jnp.int32)
    junk = jnp.full((_CHUNK,), _NPAD - 1, jnp.int32)

    @functools.partial(
        pl.kernel,
        out_type=jax.ShapeDtypeStruct((_NC, _NPAD, _F), jnp.float32),
        mesh=_sc_mesh(),
        scratch_types=[
            pltpu.VMEM((_IPT,), jnp.int32),
            pltpu.VMEM((_CHUNK,), jnp.int32),
            pltpu.VMEM((_CHUNK,), jnp.int32),
            pltpu.VMEM((_CHUNK, _F), jnp.float32),
            pltpu.VMEM((_CHUNK, _F), jnp.float32),
            pltpu.VMEM_SHARED((_NPAD, _F), jnp.float32),
            pltpu.SemaphoreType.DMA,
            pltpu.SemaphoreType.DMA,
            pltpu.SemaphoreType.DMA,
            pltpu.SemaphoreType.DMA,
        ],
    )
    def spmm_k(tab_hbm, src_hbm, dst_hbm, zeros_hbm, zidx_hbm, junk_hbm,
               out_hbm, src1d, didx_a, didx_b, rows_a, rows_b, acc,
               gsem_a, gsem_b, dsem_a, dsem_b):
        c = lax.axis_index("c")
        s = lax.axis_index("s")
        wid = c * _NS + s
        r0 = s * _RPT
        e0 = wid * _EPT

        pltpu.sync_copy(zeros_hbm, acc.at[pl.ds(r0, _RPT)])
        pltpu.sync_copy(zidx_hbm, src1d.at[pl.ds(_IPT - _CHUNK, _CHUNK)])
        pltpu.sync_copy(src_hbm.at[pl.ds(e0, _EPT)], src1d.at[pl.ds(0, _EPT)])

        def gather(j, rows, gsem):
            pltpu.async_copy(tab_hbm.at[src1d.at[pl.ds(j * _CHUNK, _CHUNK)]],
                             rows, gsem)

        def dload(j, didx, dsem):
            pltpu.async_copy(dst_hbm.at[pl.ds(e0 + j * _CHUNK, _CHUNK)],
                             didx, dsem)

        def gwait(rows, gsem):
            pltpu.make_async_copy(tab_hbm.at[pl.ds(0, _CHUNK)], rows,
                                  gsem).wait()

        def dwait(didx, dsem):
            pltpu.make_async_copy(dst_hbm.at[pl.ds(0, _CHUNK)], didx,
                                  dsem).wait()

        gather(0, rows_a, gsem_a)
        dload(0, didx_a, dsem_a)
        plsc.subcore_barrier()

        def cbody(o, carry):
            j0 = 2 * o
            gather(j0 + 1, rows_b, gsem_b)
            dload(j0 + 1, didx_b, dsem_b)
            gwait(rows_a, gsem_a)
            dwait(didx_a, dsem_a)
            pltpu.sync_copy(rows_a, acc.at[didx_a], add=True)
            gather(j0 + 2, rows_a, gsem_a)

            @pl.when(j0 + 2 < _NFULL)
            def _():
                dload(j0 + 2, didx_a, dsem_a)

            gwait(rows_b, gsem_b)
            dwait(didx_b, dsem_b)
            pltpu.sync_copy(rows_b, acc.at[didx_b], add=True)
            return carry
        lax.fori_loop(0, _NFULL // 2, cbody, 0)

        pltpu.sync_copy(junk_hbm, didx_a)
        pltpu.sync_copy(dst_hbm.at[pl.ds(e0 + _NFULL * _CHUNK, 16)],
                        didx_a.at[pl.ds(0, 16)])
        gwait(rows_a, gsem_a)
        pltpu.sync_copy(rows_a, acc.at[didx_a], add=True)

        plsc.subcore_barrier()
        pltpu.sync_copy(acc.at[pl.ds(r0, _RPT)],
                        out_hbm.at[c, pl.ds(r0, _RPT)])

    return spmm_k(table, src, dst, zeros, zidx, junk)


def _tc_pre(hists, x, w1):
    def body(h_ref, x_ref, w_ref, dis_ref, g1_ref):
        deg = h_ref[:_N, 0:1] + h_ref[_NPAD:_NPAD + _N, 0:1]
        dis = lax.rsqrt(deg + 1.0)
        dis_ref[...] = dis
        g1_ref[...] = dis * jnp.dot(x_ref[...], w_ref[...],
                                    preferred_element_type=jnp.float32)

    return pl.pallas_call(
        body,
        out_shape=(jax.ShapeDtypeStruct((_N, 1), jnp.float32),
                   jax.ShapeDtypeStruct((_N, _F), jnp.float32)),
    )(hists, x, w1)


def _tc_layer(p, g_prev, dis, b, a, gam, bet, w_next):
    def body(p_ref, g_ref, dis_ref, b_ref, a_ref, gam_ref, bet_ref, w_ref,
             out_ref):
        dis = dis_ref[...]
        ssum = p_ref[0, :_N, :] + p_ref[1, :_N, :] + g_ref[...]
        conv = dis * ssum + b_ref[...]
        h = conv * jax.nn.sigmoid(a_ref[...])
        mu = jnp.mean(h, axis=1, keepdims=True)
        d = h - mu
        var = jnp.mean(d * d, axis=1, keepdims=True)
        hn = d * lax.rsqrt(var + 1e-5) * gam_ref[...] + bet_ref[...]
        he = jnp.where(hn > 0, hn, jnp.exp(jnp.minimum(hn, 0.0)) - 1.0)
        out_ref[...] = dis * jnp.dot(he, w_ref[...],
                                     preferred_element_type=jnp.float32)

    return pl.pallas_call(
        body,
        out_shape=jax.ShapeDtypeStruct((_N, _F), jnp.float32),
    )(p, g_prev, dis, b, a, gam, bet, w_next)


def _tc_final(p, g_last, dis, b):
    def body(p_ref, g_ref, dis_ref, b_ref, out_ref):
        ssum = p_ref[0, :_N, :] + p_ref[1, :_N, :] + g_ref[...]
        out_ref[...] = dis_ref[...] * ssum + b_ref[...]

    return pl.pallas_call(
        body,
        out_shape=jax.ShapeDtypeStruct((_N, _F), jnp.float32),
    )(p, g_last, dis, b)


def kernel(x, edge_index, W1, b1, a1, g1, be1, W2, b2, a2, g2, be2,
           W3, b3, a3, g3, be3, W4, b4):
    src = edge_index[0]
    dst = edge_index[1]

    hists = _deg_hists(dst).reshape(_NC * _NPAD, _F)
    dis, h = _tc_pre(hists, x, W1)

    layer_params = [
        (b1, a1, g1, be1, W2),
        (b2, a2, g2, be2, W3),
        (b3, a3, g3, be3, W4),
    ]
    for (b, a, gam, bet, w_next) in layer_params:
        p = _spmm(h, src, dst)
        h = _tc_layer(p, h, dis, b.reshape(1, _F), a.reshape(1, 1),
                      gam.reshape(1, _F), bet.reshape(1, _F), w_next)

    p = _spmm(h, src, dst)
    return _tc_final(p, h, dis, b4.reshape(1, _F))


# speedup vs baseline: 25.7989x; 2.0128x over previous
import functools

import jax
import jax.numpy as jnp
from jax import lax
from jax.experimental import pallas as pl
from jax.experimental.pallas import tpu as pltpu
from jax.experimental.pallas import tpu_sc as plsc

_N = 10000
_E = 320000
_F = 128
_NC = 2
_NS = 16
_NW = _NC * _NS
_NPAD = 10240
_RPT = _NPAD // _NS
_EPT = _E // _NW
_CHUNK = 80
_NCH = _EPT // _CHUNK
_NB = 3


def _sc_mesh():
    return plsc.VectorSubcoreMesh(
        core_axis_name="c", subcore_axis_name="s",
        num_cores=_NC, num_subcores=_NS)


def _deg_hists(dst):
    ones = jnp.ones((_CHUNK, _F), jnp.float32)
    zeros = jnp.zeros((_RPT, _F), jnp.float32)

    @functools.partial(
        pl.kernel,
        out_type=jax.ShapeDtypeStruct((_NC, _NPAD, _F), jnp.float32),
        mesh=_sc_mesh(),
        scratch_types=[
            pltpu.VMEM((_CHUNK,), jnp.int32),
            pltpu.VMEM((_CHUNK,), jnp.int32),
            pltpu.VMEM((_CHUNK, _F), jnp.float32),
            pltpu.VMEM_SHARED((_NPAD, _F), jnp.float32),
            pltpu.SemaphoreType.DMA,
            pltpu.SemaphoreType.DMA,
        ],
    )
    def deg_k(dst_hbm, ones_hbm, zeros_hbm, hist_hbm,
              didx_a, didx_b, ones_v, hist, dsem_a, dsem_b):
        c = lax.axis_index("c")
        s = lax.axis_index("s")
        wid = c * _NS + s
        r0 = s * _RPT
        e0 = wid * _EPT

        def dload(j, didx, dsem):
            pltpu.async_copy(dst_hbm.at[pl.ds(e0 + j * _CHUNK, _CHUNK)],
                             didx, dsem)

        def dwait(didx, dsem):
            pltpu.make_async_copy(dst_hbm.at[pl.ds(0, _CHUNK)], didx,
                                  dsem).wait()

        pltpu.sync_copy(zeros_hbm, hist.at[pl.ds(r0, _RPT)])
        pltpu.sync_copy(ones_hbm, ones_v)
        dload(0, didx_a, dsem_a)
        plsc.subcore_barrier()

        def cbody(o, carry):
            j0 = 2 * o
            dload(j0 + 1, didx_b, dsem_b)
            dwait(didx_a, dsem_a)
            pltpu.sync_copy(ones_v, hist.at[didx_a], add=True)
            dload(j0 + 2, didx_a, dsem_a)
            dwait(didx_b, dsem_b)
            pltpu.sync_copy(ones_v, hist.at[didx_b], add=True)
            return carry
        lax.fori_loop(0, _NCH // 2, cbody, 0)

        dwait(didx_a, dsem_a)
        pltpu.sync_copy(ones_v, hist.at[didx_a], add=True)

        plsc.subcore_barrier()
        pltpu.sync_copy(hist.at[pl.ds(r0, _RPT)],
                        hist_hbm.at[c, pl.ds(r0, _RPT)])

    return deg_k(dst, ones, zeros)


def _spmm(table, src, dst):
    zeros = jnp.zeros((_RPT, _F), jnp.float32)

    @functools.partial(
        pl.kernel,
        out_type=jax.ShapeDtypeStruct((_NC, _NPAD, _F), jnp.float32),
        mesh=_sc_mesh(),
        scratch_types=[
            pltpu.VMEM((_EPT,), jnp.int32),
            *[pltpu.VMEM((_CHUNK,), jnp.int32)] * _NB,
            *[pltpu.VMEM((_CHUNK, _F), jnp.float32)] * _NB,
            pltpu.VMEM_SHARED((_NPAD, _F), jnp.float32),
            *[pltpu.SemaphoreType.DMA] * (3 * _NB),
        ],
    )
    def spmm_k(tab_hbm, src_hbm, dst_hbm, zeros_hbm, out_hbm,
               src1d, di0, di1, di2, ro0, ro1, ro2, acc, *sems):
        didx = [di0, di1, di2]
        rows = [ro0, ro1, ro2]
        gsem = sems[0:_NB]
        dsem = sems[_NB:2 * _NB]
        ssem = sems[2 * _NB:3 * _NB]
        c = lax.axis_index("c")
        s = lax.axis_index("s")
        wid = c * _NS + s
        r0 = s * _RPT
        e0 = wid * _EPT

        pltpu.sync_copy(zeros_hbm, acc.at[pl.ds(r0, _RPT)])
        pltpu.sync_copy(src_hbm.at[pl.ds(e0, _EPT)], src1d)

        def gather(j, b):
            pltpu.async_copy(tab_hbm.at[src1d.at[pl.ds(j * _CHUNK, _CHUNK)]],
                             rows[b], gsem[b])

        def dload(j, b):
            pltpu.async_copy(dst_hbm.at[pl.ds(e0 + j * _CHUNK, _CHUNK)],
                             didx[b], dsem[b])

        def gwait(b):
            pltpu.make_async_copy(tab_hbm.at[pl.ds(0, _CHUNK)], rows[b],
                                  gsem[b]).wait()

        def dwait(b):
            pltpu.make_async_copy(dst_hbm.at[pl.ds(0, _CHUNK)], didx[b],
                                  dsem[b]).wait()

        def swait(b):
            pltpu.make_async_copy(rows[b], acc.at[pl.ds(0, _CHUNK)],
                                  ssem[b]).wait()

        for b in range(_NB):
            gather(b, b)
            dload(b, b)
        plsc.subcore_barrier()

        def cbody(o, carry):
            j0 = _NB * o
            for b in range(_NB):
                j = j0 + b
                gwait(b)
                dwait(b)
                pltpu.async_copy(rows[b], acc.at[didx[b]], ssem[b], add=True)

                @pl.when(j + _NB < _NCH)
                def _():
                    swait(b)
                    gather(j + _NB, b)
                    dload(j + _NB, b)
            return carry
        lax.fori_loop(0, _NCH // _NB, cbody, 0)

        for (j, b) in ((123, 0), (124, 1)):
            gwait(b)
            dwait(b)
            pltpu.async_copy(rows[b], acc.at[didx[b]], ssem[b], add=True)
        for b in range(_NB):
            swait(b)

        plsc.subcore_barrier()
        pltpu.sync_copy(acc.at[pl.ds(r0, _RPT)],
                        out_hbm.at[c, pl.ds(r0, _RPT)])

    return spmm_k(table, src, dst, zeros)


def _tc_pre(hists, x, w1):
    def body(h_ref, x_ref, w_ref, dis_ref, g1_ref):
        deg = h_ref[:_N, 0:1] + h_ref[_NPAD:_NPAD + _N, 0:1]
        dis = lax.rsqrt(deg + 1.0)
        dis_ref[...] = dis
        g1_ref[...] = dis * jnp.dot(x_ref[...], w_ref[...],
                                    preferred_element_type=jnp.float32)

    return pl.pallas_call(
        body,
        out_shape=(jax.ShapeDtypeStruct((_N, 1), jnp.float32),
                   jax.ShapeDtypeStruct((_N, _F), jnp.float32)),
    )(hists, x, w1)


def _tc_layer(p, g_prev, dis, b, a, gam, bet, w_next):
    def body(p_ref, g_ref, dis_ref, b_ref, a_ref, gam_ref, bet_ref, w_ref,
             out_ref):
        dis = dis_ref[...]
        ssum = p_ref[0, :_N, :] + p_ref[1, :_N, :] + g_ref[...]
        conv = dis * ssum + b_ref[...]
        h = conv * jax.nn.sigmoid(a_ref[...])
        mu = jnp.mean(h, axis=1, keepdims=True)
        d = h - mu
        var = jnp.mean(d * d, axis=1, keepdims=True)
        hn = d * lax.rsqrt(var + 1e-5) * gam_ref[...] + bet_ref[...]
        he = jnp.where(hn > 0, hn, jnp.exp(jnp.minimum(hn, 0.0)) - 1.0)
        out_ref[...] = dis * jnp.dot(he, w_ref[...],
                                     preferred_element_type=jnp.float32)

    return pl.pallas_call(
        body,
        out_shape=jax.ShapeDtypeStruct((_N, _F), jnp.float32),
    )(p, g_prev, dis, b, a, gam, bet, w_next)


def _tc_final(p, g_last, dis, b):
    def body(p_ref, g_ref, dis_ref, b_ref, out_ref):
        ssum = p_ref[0, :_N, :] + p_ref[1, :_N, :] + g_ref[...]
        out_ref[...] = dis_ref[...] * ssum + b_ref[...]

    return pl.pallas_call(
        body,
        out_shape=jax.ShapeDtypeStruct((_N, _F), jnp.float32),
    )(p, g_last, dis, b)


def kernel(x, edge_index, W1, b1, a1, g1, be1, W2, b2, a2, g2, be2,
           W3, b3, a3, g3, be3, W4, b4):
    src = edge_index[0]
    dst = edge_index[1]

    hists = _deg_hists(dst).reshape(_NC * _NPAD, _F)
    dis, h = _tc_pre(hists, x, W1)

    layer_params = [
        (b1, a1, g1, be1, W2),
        (b2, a2, g2, be2, W3),
        (b3, a3, g3, be3, W4),
    ]
    for (b, a, gam, bet, w_next) in layer_params:
        p = _spmm(h, src, dst)
        h = _tc_layer(p, h, dis, b.reshape(1, _F), a.reshape(1, 1),
                      gam.reshape(1, _F), bet.reshape(1, _F), w_next)

    p = _spmm(h, src, dst)
    return _tc_final(p, h, dis, b4.reshape(1, _F))
